# Initial kernel scaffold; baseline (speedup 1.0000x reference)
import jax
import jax.numpy as jnp
from jax.experimental import pallas as pl


def kernel(X, A, params):
    raise NotImplementedError("write your pallas kernel here")


# speedup vs baseline: 2.6574x; 2.6574x over previous
import jax
import jax.numpy as jnp
from jax import lax
from jax.experimental import pallas as pl
import jax.experimental.pallas.tpu as pltpu

_B, _N, _F = 4, 512, 32
_COMBS = 4
_DELTA = 1.6
_EPS = 1e-3
_POOL_DIMS = [(32, 106, 181, 256), (32, 64, 96, 128), (32, 42, 53, 64)]


def _squareplus(x):
    return 0.5 * (jnp.sqrt(x * x + 4.0) + x)


def _dot(a, b):
    return jnp.dot(a, b, preferred_element_type=jnp.float32)


def _dotT(a, b):
    return lax.dot_general(a, b, (((0,), (0,)), ((), ())),
                           preferred_element_type=jnp.float32)


def _rowsum_exact(M):
    n, w = M.shape
    if w > 128:
        acc = lax.slice(M, (0, 0), (n, 128))
        for k in range(1, w // 128):
            acc = acc + lax.slice(M, (0, 128 * k), (n, 128 * (k + 1)))
    else:
        acc = M
    wa = acc.shape[1]
    a = lax.slice(acc, (0, 0), (n, 8))
    for k in range(1, wa // 8):
        a = a + lax.slice(acc, (0, 8 * k), (n, 8 * (k + 1)))
    while a.shape[1] > 1:
        h = a.shape[1] // 2
        a = lax.slice(a, (0, 0), (n, h)) + lax.slice(a, (0, h), (n, 2 * h))
    return a


def _abs_diff_sum(Xp, n):
    XpT = Xp.T

    def plane(c):
        col = lax.slice(Xp, (0, c), (n, c + 1))
        row = lax.slice(XpT, (c, 0), (c + 1, n))
        return jnp.abs(col - row)

    acc = None
    for g in range(0, _F, 8):
        p = [plane(g + k) for k in range(8)]
        l1 = [p[k] + p[k + 4] for k in range(4)]
        l2 = [l1[0] + l1[2], l1[1] + l1[3]]
        grp = l2[0] + l2[1]
        acc = grp if acc is None else acc + grp
    return acc


def _tvgnn_body(X_ref, A_ref, *refs):
    o_ref = refs[-1]
    wrefs = refs[:-1]
    w = {}
    pos = 0
    for i in range(_COMBS):
        w['W%d' % i] = wrefs[pos][...]; pos += 1
        w['b%d' % i] = wrefs[pos][...]; pos += 1
    for i in range(len(_POOL_DIMS)):
        for j in range(3):
            w['Wm%d_%d' % (i, j)] = wrefs[pos][...]; pos += 1
            w['bm%d_%d' % (i, j)] = wrefs[pos][...]; pos += 1

    X = X_ref[0]
    A = A_ref[0]
    n = _N
    for i in range(_COMBS):
        Xp = _dot(X, w['W%d' % i]) + w['b%d' % i]
        ad = _abs_diff_sum(Xp, n)
        gamma = A / jnp.maximum(ad, _EPS)
        degrees = _rowsum_exact(gamma)
        ii = lax.broadcasted_iota(jnp.int32, (n, n), 0)
        jj = lax.broadcasted_iota(jnp.int32, (n, n), 1)
        eye = jnp.where(ii == jj, 1.0, 0.0)
        diag = jnp.sum(jnp.where(ii == jj, gamma, 0.0), axis=-1,
                       keepdims=True)
        diag_vals = degrees - diag
        L = -gamma * (1.0 - eye) + eye * diag_vals
        L_adj = eye - _DELTA * L
        X = _squareplus(_dot(L_adj, Xp))
        if i != _COMBS - 1:
            h = X
            for j in range(2):
                h = _squareplus(_dot(h, w['Wm%d_%d' % (i, j)])
                                + w['bm%d_%d' % (i, j)])
            logits = _dot(h, w['Wm%d_2' % i]) + w['bm%d_2' % i]
            m = jnp.max(logits, axis=-1, keepdims=True)
            e = jnp.exp(logits - m)
            s = e * (1.0 / _rowsum_exact(e))
            Xpool = _dotT(s, X)
            t = _dot(A, s)
            Ap = _dotT(s, t)
            X, A = Xpool, Ap
            n = _POOL_DIMS[i][-1]
        else:
            o_ref[0, 0] = jnp.mean(X, axis=0)


def kernel(X, A, params):
    wlist = []
    for i in range(_COMBS):
        wlist.append(params['W%d' % i])
        wlist.append(params['b%d' % i].reshape(1, -1))
    for i in range(len(_POOL_DIMS)):
        for j in range(3):
            wlist.append(params['Wm%d_%d' % (i, j)])
            wlist.append(params['bm%d_%d' % (i, j)].reshape(1, -1))

    in_specs = [
        pl.BlockSpec((1, _N, _F), lambda b: (b, 0, 0)),
        pl.BlockSpec((1, _N, _N), lambda b: (b, 0, 0)),
    ]
    for wv in wlist:
        in_specs.append(pl.BlockSpec(wv.shape, lambda b: (0, 0)))

    out = pl.pallas_call(
        _tvgnn_body,
        grid=(_B,),
        in_specs=in_specs,
        out_specs=pl.BlockSpec((1, 1, _F), lambda b: (b, 0, 0)),
        out_shape=jax.ShapeDtypeStruct((_B, 1, _F), jnp.float32),
        compiler_params=pltpu.CompilerParams(
            dimension_semantics=("arbitrary",)),
    )(X, A, *wlist)
    return out.reshape(_B, _F)
